# Initial kernel scaffold; baseline (speedup 1.0000x reference)
import jax
import jax.numpy as jnp
from jax.experimental import pallas as pl


def kernel(ui_x, ui_edge_index, s_x, s_edge_index, k_x, k_edge_index, a, W_l_ui, b_l_ui, W_r_ui, W_l_s, b_l_s, W_r_s, W_l_k, b_l_k, W_r_k):
    raise NotImplementedError("write your pallas kernel here")


# speedup vs baseline: 4.2573x; 4.2573x over previous
import functools

import jax
import jax.numpy as jnp
from jax import lax
from jax.experimental import pallas as pl
from jax.experimental.pallas import tpu as pltpu
from jax.experimental.pallas import tpu_sc as plsc

N = 10000
E = 320000
D = 128
DA = 144
NC, NS = 2, 16
CB = 80
EPS = E // NC
EPT = EPS // NS
RPT = N // NS
BN = 2000


def _mm_body(x_ref, wl_ref, wr_ref, y_ref, z_ref):
    x = x_ref[0]
    y_ref[0] = jnp.dot(x, wl_ref[0], preferred_element_type=jnp.float32)
    z_ref[0] = jnp.dot(x, wr_ref[0], preferred_element_type=jnp.float32)


def _matmuls(xs, wlts, wrts):
    return pl.pallas_call(
        _mm_body,
        grid=(3, N // BN),
        in_specs=[
            pl.BlockSpec((1, BN, D), lambda g, i: (g, i, 0)),
            pl.BlockSpec((1, D, D), lambda g, i: (g, 0, 0)),
            pl.BlockSpec((1, D, D), lambda g, i: (g, 0, 0)),
        ],
        out_specs=[
            pl.BlockSpec((1, BN, D), lambda g, i: (g, i, 0)),
            pl.BlockSpec((1, BN, D), lambda g, i: (g, i, 0)),
        ],
        out_shape=[jax.ShapeDtypeStruct((3, N, D), jnp.float32)] * 2,
    )(xs, wlts, wrts)


def _sc_body(y0, y1, y2, e0, e1, e2, zrows, out0, out1, out2,
             acc, src_v, dst_v, rows_v, sem):
    c = lax.axis_index("c")
    s = lax.axis_index("s")
    r0 = s * RPT
    for y_hbm, e_hbm, o_hbm in ((y0, e0, out0), (y1, e1, out1), (y2, e2, out2)):
        pltpu.sync_copy(zrows, acc.at[pl.ds(r0, RPT)])
        plsc.subcore_barrier()
        ebase = c * EPS + s * EPT

        def body(i, carry, e_hbm=e_hbm, y_hbm=y_hbm, ebase=ebase):
            base = ebase + i * CB
            pltpu.sync_copy(e_hbm.at[0, pl.ds(base, CB)], src_v)
            pltpu.sync_copy(e_hbm.at[1, pl.ds(base, CB)], dst_v)
            pltpu.async_copy(y_hbm.at[src_v], rows_v, sem).wait()
            pltpu.sync_copy(rows_v, acc.at[dst_v], add=True)
            return carry

        lax.fori_loop(0, EPT // CB, body, 0)
        plsc.subcore_barrier()
        pltpu.sync_copy(acc.at[pl.ds(r0, RPT)], o_hbm.at[c, pl.ds(r0, RPT)])
        plsc.subcore_barrier()


@functools.cache
def _sc_call():
    return pl.kernel(
        _sc_body,
        out_type=[jax.ShapeDtypeStruct((NC, N, DA), jnp.float32)] * 3,
        mesh=plsc.VectorSubcoreMesh(
            core_axis_name="c", subcore_axis_name="s",
            num_cores=NC, num_subcores=NS),
        scratch_types=[
            pltpu.VMEM_SHARED((N, DA), jnp.float32),
            pltpu.VMEM((CB,), jnp.int32),
            pltpu.VMEM((CB,), jnp.int32),
            pltpu.VMEM((CB, DA), jnp.float32),
            pltpu.SemaphoreType.DMA,
        ],
        compiler_params=pltpu.CompilerParams(use_tc_tiling_on_sc=False),
    )


def _comb_body(a_ref, acc0_ref, acc1_ref, acc2_ref, z_ref, b_ref, out_ref):
    av = a_ref[0, 0]
    weights = (1.0, av, 1.0 - av)
    tot = None
    for g, acc_ref in enumerate((acc0_ref, acc1_ref, acc2_ref)):
        sm = acc_ref[0] + acc_ref[1]
        cnt = sm[:, D:D + 1]
        mean = sm[:, :D] / jnp.maximum(cnt, 1.0)
        t = mean + b_ref[g][None, :] + z_ref[g]
        tot = t * weights[g] if tot is None else tot + t * weights[g]
    out_ref[...] = tot


def _combine(a2d, acc0, acc1, acc2, z, bs):
    accspec = pl.BlockSpec((NC, BN, DA), lambda i: (0, i, 0))
    return pl.pallas_call(
        _comb_body,
        grid=(N // BN,),
        in_specs=[
            pl.BlockSpec(memory_space=pltpu.SMEM),
            accspec, accspec, accspec,
            pl.BlockSpec((3, BN, D), lambda i: (0, i, 0)),
            pl.BlockSpec((3, D), lambda i: (0, 0)),
        ],
        out_specs=pl.BlockSpec((BN, D), lambda i: (i, 0)),
        out_shape=jax.ShapeDtypeStruct((N, D), jnp.float32),
    )(a2d, acc0, acc1, acc2, z, bs)


def kernel(ui_x, ui_edge_index, s_x, s_edge_index, k_x, k_edge_index, a,
           W_l_ui, b_l_ui, W_r_ui, W_l_s, b_l_s, W_r_s, W_l_k, b_l_k, W_r_k):
    xs = jnp.stack([ui_x, s_x, k_x])
    wlts = jnp.stack([W_l_ui.T, W_l_s.T, W_l_k.T])
    wrts = jnp.stack([W_r_ui.T, W_r_s.T, W_r_k.T])
    y, z = _matmuls(xs, wlts, wrts)
    aug = jnp.concatenate(
        [y, jnp.ones((3, N, 1), jnp.float32),
         jnp.zeros((3, N, DA - D - 1), jnp.float32)], axis=2)
    zrows = jnp.zeros((RPT, DA), jnp.float32)
    acc0, acc1, acc2 = _sc_call()(
        aug[0], aug[1], aug[2],
        ui_edge_index, s_edge_index, k_edge_index, zrows)
    bs = jnp.stack([b_l_ui, b_l_s, b_l_k])
    a2d = jnp.reshape(a, (1, 1))
    return _combine(a2d, acc0, acc1, acc2, z, bs)


# speedup vs baseline: 6.7189x; 1.5782x over previous
import functools

import jax
import jax.numpy as jnp
from jax import lax
from jax.experimental import pallas as pl
from jax.experimental.pallas import tpu as pltpu
from jax.experimental.pallas import tpu_sc as plsc

N = 10000
E = 320000
D = 128
DA = 144
NC, NS = 2, 16
CB = 100
NCH = E // (NC * NS * CB)
HCH = NCH // 2
RPT = N // NS
BN = 2000


def _mm_body(x_ref, wl_ref, wr_ref, y_ref, z_ref):
    x = x_ref[0]
    y_ref[0] = jnp.dot(x, wl_ref[0], preferred_element_type=jnp.float32)
    z_ref[0] = jnp.dot(x, wr_ref[0], preferred_element_type=jnp.float32)


def _matmuls(xs, wlts, wrts):
    return pl.pallas_call(
        _mm_body,
        grid=(3, N // BN),
        in_specs=[
            pl.BlockSpec((1, BN, D), lambda g, i: (g, i, 0)),
            pl.BlockSpec((1, D, D), lambda g, i: (g, 0, 0)),
            pl.BlockSpec((1, D, D), lambda g, i: (g, 0, 0)),
        ],
        out_specs=[
            pl.BlockSpec((1, BN, D), lambda g, i: (g, i, 0)),
            pl.BlockSpec((1, BN, D), lambda g, i: (g, i, 0)),
        ],
        out_shape=[jax.ShapeDtypeStruct((3, N, D), jnp.float32)] * 2,
    )(xs, wlts, wrts)


def _sc_body(y0, y1, y2, e0, e1, e2, zrows, out0, out1, out2,
             acc, srcb, dstb, rows0, rows1, gs0, gs1, ss0, ss1):
    c = lax.axis_index("c")
    s = lax.axis_index("s")
    w = c * NS + s
    r0 = s * RPT
    rbase = w * NCH
    for y_hbm, e_hbm, o_hbm in ((y0, e0, out0), (y1, e1, out1), (y2, e2, out2)):
        pltpu.sync_copy(zrows, acc.at[pl.ds(r0, RPT)])
        plsc.subcore_barrier()

        for h in range(NCH // HCH):
            hbase = rbase + h * HCH
            pltpu.sync_copy(e_hbm.at[0, pl.ds(hbase, HCH), :], srcb)
            pltpu.sync_copy(e_hbm.at[1, pl.ds(hbase, HCH), :], dstb)

            pltpu.async_copy(y_hbm.at[srcb.at[0]], rows0, gs0)
            pltpu.async_copy(y_hbm.at[srcb.at[1]], rows1, gs1)

            def body(i, carry, y_hbm=y_hbm):
                i0 = 2 * i
                pltpu.make_async_copy(y_hbm.at[srcb.at[i0]], rows0, gs0).wait()
                sc0 = pltpu.async_copy(rows0, acc.at[dstb.at[i0]], ss0,
                                       add=True)
                pltpu.make_async_copy(
                    y_hbm.at[srcb.at[i0 + 1]], rows1, gs1).wait()
                sc1 = pltpu.async_copy(rows1, acc.at[dstb.at[i0 + 1]], ss1,
                                       add=True)
                sc0.wait()

                @pl.when(i < HCH // 2 - 1)
                def _():
                    pltpu.async_copy(y_hbm.at[srcb.at[i0 + 2]], rows0, gs0)

                sc1.wait()

                @pl.when(i < HCH // 2 - 1)
                def _():
                    pltpu.async_copy(y_hbm.at[srcb.at[i0 + 3]], rows1, gs1)

                return carry

            lax.fori_loop(0, HCH // 2, body, 0)
        plsc.subcore_barrier()
        pltpu.sync_copy(acc.at[pl.ds(r0, RPT)], o_hbm.at[c, pl.ds(r0, RPT)])
        plsc.subcore_barrier()


@functools.cache
def _sc_call():
    return pl.kernel(
        _sc_body,
        out_type=[jax.ShapeDtypeStruct((NC, N, DA), jnp.float32)] * 3,
        mesh=plsc.VectorSubcoreMesh(
            core_axis_name="c", subcore_axis_name="s",
            num_cores=NC, num_subcores=NS),
        scratch_types=[
            pltpu.VMEM_SHARED((N, DA), jnp.float32),
            pltpu.VMEM((HCH, CB), jnp.int32),
            pltpu.VMEM((HCH, CB), jnp.int32),
            pltpu.VMEM((CB, DA), jnp.float32),
            pltpu.VMEM((CB, DA), jnp.float32),
            pltpu.SemaphoreType.DMA,
            pltpu.SemaphoreType.DMA,
            pltpu.SemaphoreType.DMA,
            pltpu.SemaphoreType.DMA,
        ],
        compiler_params=pltpu.CompilerParams(use_tc_tiling_on_sc=False),
    )


def _comb_body(a_ref, acc0_ref, acc1_ref, acc2_ref, z_ref, b_ref, out_ref):
    av = a_ref[0, 0]
    weights = (1.0, av, 1.0 - av)
    tot = None
    for g, acc_ref in enumerate((acc0_ref, acc1_ref, acc2_ref)):
        sm = acc_ref[0] + acc_ref[1]
        cnt = sm[:, D:D + 1]
        mean = sm[:, :D] / jnp.maximum(cnt, 1.0)
        t = mean + b_ref[g][None, :] + z_ref[g]
        tot = t * weights[g] if tot is None else tot + t * weights[g]
    out_ref[...] = tot


def _combine(a2d, acc0, acc1, acc2, z, bs):
    accspec = pl.BlockSpec((NC, BN, DA), lambda i: (0, i, 0))
    return pl.pallas_call(
        _comb_body,
        grid=(N // BN,),
        in_specs=[
            pl.BlockSpec(memory_space=pltpu.SMEM),
            accspec, accspec, accspec,
            pl.BlockSpec((3, BN, D), lambda i: (0, i, 0)),
            pl.BlockSpec((3, D), lambda i: (0, 0)),
        ],
        out_specs=pl.BlockSpec((BN, D), lambda i: (i, 0)),
        out_shape=jax.ShapeDtypeStruct((N, D), jnp.float32),
    )(a2d, acc0, acc1, acc2, z, bs)


def kernel(ui_x, ui_edge_index, s_x, s_edge_index, k_x, k_edge_index, a,
           W_l_ui, b_l_ui, W_r_ui, W_l_s, b_l_s, W_r_s, W_l_k, b_l_k, W_r_k):
    xs = jnp.stack([ui_x, s_x, k_x])
    wlts = jnp.stack([W_l_ui.T, W_l_s.T, W_l_k.T])
    wrts = jnp.stack([W_r_ui.T, W_r_s.T, W_r_k.T])
    y, z = _matmuls(xs, wlts, wrts)
    aug = jnp.concatenate(
        [y, jnp.ones((3, N, 1), jnp.float32),
         jnp.zeros((3, N, DA - D - 1), jnp.float32)], axis=2)
    zrows = jnp.zeros((RPT, DA), jnp.float32)
    acc0, acc1, acc2 = _sc_call()(
        aug[0], aug[1], aug[2],
        ui_edge_index.reshape(2, E // CB, CB),
        s_edge_index.reshape(2, E // CB, CB),
        k_edge_index.reshape(2, E // CB, CB), zrows)
    bs = jnp.stack([b_l_ui, b_l_s, b_l_k])
    a2d = jnp.reshape(a, (1, 1))
    return _combine(a2d, acc0, acc1, acc2, z, bs)


# speedup vs baseline: 7.4502x; 1.1088x over previous
import functools

import jax
import jax.numpy as jnp
from jax import lax
from jax.experimental import pallas as pl
from jax.experimental.pallas import tpu as pltpu
from jax.experimental.pallas import tpu_sc as plsc

N = 10000
E = 320000
D = 128
DA = 144
NC, NS = 2, 16
CB = 50
NB = 4
NCH = E // (NC * NS * CB)
HCH = NCH // 2
RPT = N // NS
BN = 2000


def _mm_body(x_ref, wl_ref, wr_ref, y_ref, z_ref):
    x = x_ref[0]
    y_ref[0] = jnp.dot(x, wl_ref[0], preferred_element_type=jnp.float32)
    z_ref[0] = jnp.dot(x, wr_ref[0], preferred_element_type=jnp.float32)


def _matmuls(xs, wlts, wrts):
    return pl.pallas_call(
        _mm_body,
        grid=(3, N // BN),
        in_specs=[
            pl.BlockSpec((1, BN, D), lambda g, i: (g, i, 0)),
            pl.BlockSpec((1, D, D), lambda g, i: (g, 0, 0)),
            pl.BlockSpec((1, D, D), lambda g, i: (g, 0, 0)),
        ],
        out_specs=[
            pl.BlockSpec((1, BN, D), lambda g, i: (g, i, 0)),
            pl.BlockSpec((1, BN, D), lambda g, i: (g, i, 0)),
        ],
        out_shape=[jax.ShapeDtypeStruct((3, N, D), jnp.float32)] * 2,
    )(xs, wlts, wrts)


def _sc_body(y0, y1, y2, e0, e1, e2, zrows, out0, out1, out2,
             acc, srcb, dstb, *bufs_and_sems):
    bufs = bufs_and_sems[:NB]
    gsems = bufs_and_sems[NB:2 * NB]
    ssems = bufs_and_sems[2 * NB:3 * NB]
    c = lax.axis_index("c")
    s = lax.axis_index("s")
    w = c * NS + s
    r0 = s * RPT
    rbase = w * NCH
    for y_hbm, e_hbm, o_hbm in ((y0, e0, out0), (y1, e1, out1), (y2, e2, out2)):
        pltpu.sync_copy(zrows, acc.at[pl.ds(r0, RPT)])
        plsc.subcore_barrier()

        for h in range(NCH // HCH):
            hbase = rbase + h * HCH
            pltpu.sync_copy(e_hbm.at[0, pl.ds(hbase, HCH), :], srcb)
            pltpu.sync_copy(e_hbm.at[1, pl.ds(hbase, HCH), :], dstb)

            for j in range(NB):
                pltpu.async_copy(y_hbm.at[srcb.at[j]], bufs[j], gsems[j])

            def body(i, carry, y_hbm=y_hbm):
                base = NB * i
                scs = []
                for j in range(NB):
                    pltpu.make_async_copy(
                        y_hbm.at[srcb.at[base + j]], bufs[j], gsems[j]).wait()
                    scs.append(pltpu.async_copy(
                        bufs[j], acc.at[dstb.at[base + j]], ssems[j],
                        add=True))
                for j in range(NB):
                    scs[j].wait()

                    @pl.when(i < HCH // NB - 1)
                    def _(j=j):
                        pltpu.async_copy(
                            y_hbm.at[srcb.at[base + NB + j]], bufs[j],
                            gsems[j])

                return carry

            lax.fori_loop(0, HCH // NB, body, 0)
        plsc.subcore_barrier()
        pltpu.sync_copy(acc.at[pl.ds(r0, RPT)], o_hbm.at[c, pl.ds(r0, RPT)])
        plsc.subcore_barrier()


@functools.cache
def _sc_call():
    return pl.kernel(
        _sc_body,
        out_type=[jax.ShapeDtypeStruct((NC, N, DA), jnp.float32)] * 3,
        mesh=plsc.VectorSubcoreMesh(
            core_axis_name="c", subcore_axis_name="s",
            num_cores=NC, num_subcores=NS),
        scratch_types=[
            pltpu.VMEM_SHARED((N, DA), jnp.float32),
            pltpu.VMEM((HCH, CB), jnp.int32),
            pltpu.VMEM((HCH, CB), jnp.int32),
        ] + [pltpu.VMEM((CB, DA), jnp.float32)] * NB
          + [pltpu.SemaphoreType.DMA] * (2 * NB),
        compiler_params=pltpu.CompilerParams(use_tc_tiling_on_sc=False),
    )


def _comb_body(a_ref, acc0_ref, acc1_ref, acc2_ref, z_ref, b_ref, out_ref):
    av = a_ref[0, 0]
    weights = (1.0, av, 1.0 - av)
    tot = None
    for g, acc_ref in enumerate((acc0_ref, acc1_ref, acc2_ref)):
        sm = acc_ref[0] + acc_ref[1]
        cnt = sm[:, D:D + 1]
        mean = sm[:, :D] / jnp.maximum(cnt, 1.0)
        t = mean + b_ref[g][None, :] + z_ref[g]
        tot = t * weights[g] if tot is None else tot + t * weights[g]
    out_ref[...] = tot


def _combine(a2d, acc0, acc1, acc2, z, bs):
    accspec = pl.BlockSpec((NC, BN, DA), lambda i: (0, i, 0))
    return pl.pallas_call(
        _comb_body,
        grid=(N // BN,),
        in_specs=[
            pl.BlockSpec(memory_space=pltpu.SMEM),
            accspec, accspec, accspec,
            pl.BlockSpec((3, BN, D), lambda i: (0, i, 0)),
            pl.BlockSpec((3, D), lambda i: (0, 0)),
        ],
        out_specs=pl.BlockSpec((BN, D), lambda i: (i, 0)),
        out_shape=jax.ShapeDtypeStruct((N, D), jnp.float32),
    )(a2d, acc0, acc1, acc2, z, bs)


def kernel(ui_x, ui_edge_index, s_x, s_edge_index, k_x, k_edge_index, a,
           W_l_ui, b_l_ui, W_r_ui, W_l_s, b_l_s, W_r_s, W_l_k, b_l_k, W_r_k):
    xs = jnp.stack([ui_x, s_x, k_x])
    wlts = jnp.stack([W_l_ui.T, W_l_s.T, W_l_k.T])
    wrts = jnp.stack([W_r_ui.T, W_r_s.T, W_r_k.T])
    y, z = _matmuls(xs, wlts, wrts)
    aug = jnp.concatenate(
        [y, jnp.ones((3, N, 1), jnp.float32),
         jnp.zeros((3, N, DA - D - 1), jnp.float32)], axis=2)
    zrows = jnp.zeros((RPT, DA), jnp.float32)
    acc0, acc1, acc2 = _sc_call()(
        aug[0], aug[1], aug[2],
        ui_edge_index.reshape(2, E // CB, CB),
        s_edge_index.reshape(2, E // CB, CB),
        k_edge_index.reshape(2, E // CB, CB), zrows)
    bs = jnp.stack([b_l_ui, b_l_s, b_l_k])
    a2d = jnp.reshape(a, (1, 1))
    return _combine(a2d, acc0, acc1, acc2, z, bs)


# speedup vs baseline: 8.0486x; 1.0803x over previous
import functools

import jax
import jax.numpy as jnp
from jax import lax
from jax.experimental import pallas as pl
from jax.experimental.pallas import tpu as pltpu
from jax.experimental.pallas import tpu_sc as plsc

N = 10000
E = 320000
D = 128
DA = 144
NC, NS = 2, 16
CB = 50
NB = 4
NCH = E // (NC * NS * CB)
HCH = NCH // 2
RPT = N // NS
BN = 2000


def _mm_body(x0_ref, x1_ref, x2_ref, wl_ref, wr_ref,
             a0_ref, a1_ref, a2_ref, z_ref):
    onehot = (lax.broadcasted_iota(jnp.int32, (BN, DA - D), 1) == 0)
    pad = jnp.where(onehot, 1.0, 0.0).astype(jnp.float32)
    for g, (x_ref, a_ref) in enumerate(
            ((x0_ref, a0_ref), (x1_ref, a1_ref), (x2_ref, a2_ref))):
        x = x_ref[...]
        a_ref[:, 0:D] = jnp.dot(x, wl_ref[g], preferred_element_type=jnp.float32)
        a_ref[:, D:DA] = pad
        z_ref[g] = jnp.dot(x, wr_ref[g], preferred_element_type=jnp.float32)


def _matmuls(x0, x1, x2, wlts, wrts):
    xspec = pl.BlockSpec((BN, D), lambda i: (i, 0))
    wspec = pl.BlockSpec((3, D, D), lambda i: (0, 0, 0))
    aspec = pl.BlockSpec((BN, DA), lambda i: (i, 0))
    return pl.pallas_call(
        _mm_body,
        grid=(N // BN,),
        in_specs=[xspec, xspec, xspec, wspec, wspec],
        out_specs=[aspec, aspec, aspec,
                   pl.BlockSpec((3, BN, D), lambda i: (0, i, 0))],
        out_shape=[jax.ShapeDtypeStruct((N, DA), jnp.float32)] * 3
                  + [jax.ShapeDtypeStruct((3, N, D), jnp.float32)],
    )(x0, x1, x2, wlts, wrts)


def _sc_body(y0, y1, y2, e0, e1, e2, zrows, out0, out1, out2,
             acc, srcb, dstb, *bufs_and_sems):
    bufs = bufs_and_sems[:NB]
    gsems = bufs_and_sems[NB:2 * NB]
    ssems = bufs_and_sems[2 * NB:3 * NB]
    c = lax.axis_index("c")
    s = lax.axis_index("s")
    w = c * NS + s
    r0 = s * RPT
    rbase = w * NCH
    for y_hbm, e_hbm, o_hbm in ((y0, e0, out0), (y1, e1, out1), (y2, e2, out2)):
        pltpu.sync_copy(zrows, acc.at[pl.ds(r0, RPT)])
        plsc.subcore_barrier()

        for h in range(NCH // HCH):
            hbase = rbase + h * HCH
            pltpu.sync_copy(e_hbm.at[0, pl.ds(hbase, HCH), :], srcb)
            pltpu.sync_copy(e_hbm.at[1, pl.ds(hbase, HCH), :], dstb)

            for j in range(NB):
                pltpu.async_copy(y_hbm.at[srcb.at[j]], bufs[j], gsems[j])

            def body(i, carry, y_hbm=y_hbm):
                base = NB * i
                scs = []
                for j in range(NB):
                    pltpu.make_async_copy(
                        y_hbm.at[srcb.at[base + j]], bufs[j], gsems[j]).wait()
                    scs.append(pltpu.async_copy(
                        bufs[j], acc.at[dstb.at[base + j]], ssems[j],
                        add=True))
                for j in range(NB):
                    scs[j].wait()

                    @pl.when(i < HCH // NB - 1)
                    def _(j=j):
                        pltpu.async_copy(
                            y_hbm.at[srcb.at[base + NB + j]], bufs[j],
                            gsems[j])

                return carry

            lax.fori_loop(0, HCH // NB, body, 0)
        plsc.subcore_barrier()
        pltpu.sync_copy(acc.at[pl.ds(r0, RPT)], o_hbm.at[c, pl.ds(r0, RPT)])
        plsc.subcore_barrier()


@functools.cache
def _sc_call():
    return pl.kernel(
        _sc_body,
        out_type=[jax.ShapeDtypeStruct((NC, N, DA), jnp.float32)] * 3,
        mesh=plsc.VectorSubcoreMesh(
            core_axis_name="c", subcore_axis_name="s",
            num_cores=NC, num_subcores=NS),
        scratch_types=[
            pltpu.VMEM_SHARED((N, DA), jnp.float32),
            pltpu.VMEM((HCH, CB), jnp.int32),
            pltpu.VMEM((HCH, CB), jnp.int32),
        ] + [pltpu.VMEM((CB, DA), jnp.float32)] * NB
          + [pltpu.SemaphoreType.DMA] * (2 * NB),
        compiler_params=pltpu.CompilerParams(use_tc_tiling_on_sc=False),
    )


def _comb_body(a_ref, acc0_ref, acc1_ref, acc2_ref, z_ref, b_ref, out_ref):
    av = a_ref[0, 0]
    weights = (1.0, av, 1.0 - av)
    tot = None
    for g, acc_ref in enumerate((acc0_ref, acc1_ref, acc2_ref)):
        sm = acc_ref[0] + acc_ref[1]
        cnt = sm[:, D:D + 1]
        mean = sm[:, :D] / jnp.maximum(cnt, 1.0)
        t = mean + b_ref[g][None, :] + z_ref[g]
        tot = t * weights[g] if tot is None else tot + t * weights[g]
    out_ref[...] = tot


def _combine(a2d, acc0, acc1, acc2, z, bs):
    accspec = pl.BlockSpec((NC, BN, DA), lambda i: (0, i, 0))
    return pl.pallas_call(
        _comb_body,
        grid=(N // BN,),
        in_specs=[
            pl.BlockSpec(memory_space=pltpu.SMEM),
            accspec, accspec, accspec,
            pl.BlockSpec((3, BN, D), lambda i: (0, i, 0)),
            pl.BlockSpec((3, D), lambda i: (0, 0)),
        ],
        out_specs=pl.BlockSpec((BN, D), lambda i: (i, 0)),
        out_shape=jax.ShapeDtypeStruct((N, D), jnp.float32),
    )(a2d, acc0, acc1, acc2, z, bs)


def kernel(ui_x, ui_edge_index, s_x, s_edge_index, k_x, k_edge_index, a,
           W_l_ui, b_l_ui, W_r_ui, W_l_s, b_l_s, W_r_s, W_l_k, b_l_k, W_r_k):
    wlts = jnp.stack([W_l_ui.T, W_l_s.T, W_l_k.T])
    wrts = jnp.stack([W_r_ui.T, W_r_s.T, W_r_k.T])
    aug0, aug1, aug2, z = _matmuls(ui_x, s_x, k_x, wlts, wrts)
    zrows = jnp.zeros((RPT, DA), jnp.float32)
    acc0, acc1, acc2 = _sc_call()(
        aug0, aug1, aug2,
        ui_edge_index.reshape(2, E // CB, CB),
        s_edge_index.reshape(2, E // CB, CB),
        k_edge_index.reshape(2, E // CB, CB), zrows)
    bs = jnp.stack([b_l_ui, b_l_s, b_l_k])
    a2d = jnp.reshape(a, (1, 1))
    return _combine(a2d, acc0, acc1, acc2, z, bs)


# speedup vs baseline: 9.0823x; 1.1284x over previous
import functools

import jax
import jax.numpy as jnp
from jax import lax
from jax.experimental import pallas as pl
from jax.experimental.pallas import tpu as pltpu
from jax.experimental.pallas import tpu_sc as plsc

N = 10000
E = 320000
D = 128
DA = 144
NC, NS = 2, 16
CB = 40
NB = 5
NCH = E // (NC * NS * CB)
HCH = NCH // 2
RPT = N // NS
BN = 2000


def _mm_body(x0_ref, x1_ref, x2_ref, wl_ref, wr_ref,
             a0_ref, a1_ref, a2_ref, z_ref):
    onehot = (lax.broadcasted_iota(jnp.int32, (BN, DA - D), 1) == 0)
    pad = jnp.where(onehot, 1.0, 0.0).astype(jnp.float32)
    for g, (x_ref, a_ref) in enumerate(
            ((x0_ref, a0_ref), (x1_ref, a1_ref), (x2_ref, a2_ref))):
        x = x_ref[...]
        a_ref[:, 0:D] = jnp.dot(x, wl_ref[g], preferred_element_type=jnp.float32)
        a_ref[:, D:DA] = pad
        z_ref[g] = jnp.dot(x, wr_ref[g], preferred_element_type=jnp.float32)


def _matmuls(x0, x1, x2, wlts, wrts):
    xspec = pl.BlockSpec((BN, D), lambda i: (i, 0))
    wspec = pl.BlockSpec((3, D, D), lambda i: (0, 0, 0))
    aspec = pl.BlockSpec((BN, DA), lambda i: (i, 0))
    return pl.pallas_call(
        _mm_body,
        grid=(N // BN,),
        in_specs=[xspec, xspec, xspec, wspec, wspec],
        out_specs=[aspec, aspec, aspec,
                   pl.BlockSpec((3, BN, D), lambda i: (0, i, 0))],
        out_shape=[jax.ShapeDtypeStruct((N, DA), jnp.float32)] * 3
                  + [jax.ShapeDtypeStruct((3, N, D), jnp.float32)],
    )(x0, x1, x2, wlts, wrts)


def _sc_body(y0, y1, y2, e0, e1, e2, zrows, out0, out1, out2,
             acc, srcb, dstb, *bufs_and_sems):
    bufs = bufs_and_sems[:NB]
    gsems = bufs_and_sems[NB:2 * NB]
    ssems = bufs_and_sems[2 * NB:3 * NB]
    c = lax.axis_index("c")
    s = lax.axis_index("s")
    w = c * NS + s
    r0 = s * RPT
    rbase = w * NCH
    for y_hbm, e_hbm, o_hbm in ((y0, e0, out0), (y1, e1, out1), (y2, e2, out2)):
        pltpu.sync_copy(zrows, acc.at[pl.ds(r0, RPT)])
        plsc.subcore_barrier()

        for h in range(NCH // HCH):
            hbase = rbase + h * HCH
            pltpu.sync_copy(e_hbm.at[0, pl.ds(hbase, HCH), :], srcb)
            pltpu.sync_copy(e_hbm.at[1, pl.ds(hbase, HCH), :], dstb)

            for j in range(NB):
                pltpu.async_copy(y_hbm.at[srcb.at[j]], bufs[j], gsems[j])

            def body(i, carry, y_hbm=y_hbm):
                base = NB * i
                scs = []
                for j in range(NB):
                    pltpu.make_async_copy(
                        y_hbm.at[srcb.at[base + j]], bufs[j], gsems[j]).wait()
                    scs.append(pltpu.async_copy(
                        bufs[j], acc.at[dstb.at[base + j]], ssems[j],
                        add=True))
                for j in range(NB):
                    scs[j].wait()

                    @pl.when(i < HCH // NB - 1)
                    def _(j=j):
                        pltpu.async_copy(
                            y_hbm.at[srcb.at[base + NB + j]], bufs[j],
                            gsems[j])

                return carry

            lax.fori_loop(0, HCH // NB, body, 0)
        plsc.subcore_barrier()
        pltpu.sync_copy(acc.at[pl.ds(r0, RPT)], o_hbm.at[c, pl.ds(r0, RPT)])
        plsc.subcore_barrier()


@functools.cache
def _sc_call():
    return pl.kernel(
        _sc_body,
        out_type=[jax.ShapeDtypeStruct((NC, N, DA), jnp.float32)] * 3,
        mesh=plsc.VectorSubcoreMesh(
            core_axis_name="c", subcore_axis_name="s",
            num_cores=NC, num_subcores=NS),
        scratch_types=[
            pltpu.VMEM_SHARED((N, DA), jnp.float32),
            pltpu.VMEM((HCH, CB), jnp.int32),
            pltpu.VMEM((HCH, CB), jnp.int32),
        ] + [pltpu.VMEM((CB, DA), jnp.float32)] * NB
          + [pltpu.SemaphoreType.DMA] * (2 * NB),
        compiler_params=pltpu.CompilerParams(use_tc_tiling_on_sc=False),
    )


def _comb_body(a_ref, acc0_ref, acc1_ref, acc2_ref, z_ref, b_ref, out_ref):
    av = a_ref[0, 0]
    weights = (1.0, av, 1.0 - av)
    tot = None
    for g, acc_ref in enumerate((acc0_ref, acc1_ref, acc2_ref)):
        sm = acc_ref[0] + acc_ref[1]
        cnt = sm[:, D:D + 1]
        mean = sm[:, :D] / jnp.maximum(cnt, 1.0)
        t = mean + b_ref[g][None, :] + z_ref[g]
        tot = t * weights[g] if tot is None else tot + t * weights[g]
    out_ref[...] = tot


def _combine(a2d, acc0, acc1, acc2, z, bs):
    accspec = pl.BlockSpec((NC, BN, DA), lambda i: (0, i, 0))
    return pl.pallas_call(
        _comb_body,
        grid=(N // BN,),
        in_specs=[
            pl.BlockSpec(memory_space=pltpu.SMEM),
            accspec, accspec, accspec,
            pl.BlockSpec((3, BN, D), lambda i: (0, i, 0)),
            pl.BlockSpec((3, D), lambda i: (0, 0)),
        ],
        out_specs=pl.BlockSpec((BN, D), lambda i: (i, 0)),
        out_shape=jax.ShapeDtypeStruct((N, D), jnp.float32),
    )(a2d, acc0, acc1, acc2, z, bs)


def kernel(ui_x, ui_edge_index, s_x, s_edge_index, k_x, k_edge_index, a,
           W_l_ui, b_l_ui, W_r_ui, W_l_s, b_l_s, W_r_s, W_l_k, b_l_k, W_r_k):
    wlts = jnp.stack([W_l_ui.T, W_l_s.T, W_l_k.T])
    wrts = jnp.stack([W_r_ui.T, W_r_s.T, W_r_k.T])
    aug0, aug1, aug2, z = _matmuls(ui_x, s_x, k_x, wlts, wrts)
    zrows = jnp.zeros((RPT, DA), jnp.float32)
    acc0, acc1, acc2 = _sc_call()(
        aug0, aug1, aug2,
        ui_edge_index.reshape(2, E // CB, CB),
        s_edge_index.reshape(2, E // CB, CB),
        k_edge_index.reshape(2, E // CB, CB), zrows)
    bs = jnp.stack([b_l_ui, b_l_s, b_l_k])
    a2d = jnp.reshape(a, (1, 1))
    return _combine(a2d, acc0, acc1, acc2, z, bs)
